# baseline (speedup 1.0000x reference)
import jax
import jax.numpy as jnp
from jax.experimental import pallas as pl
from jax.experimental.pallas import tpu as pltpu

_G = 4
_S2_STARTS = (0, 24, 48, 64)

_VMEM = pl.BlockSpec(memory_space=pltpu.MemorySpace.VMEM)


def _fused_kernel(x_ref, bkv_ref, l1_ref, g1h0_ref, g1h1_ref, bm1_ref,
                  l2_ref, g2c_ref, bm2_ref,
                  l3_ref, g3_ref, bm3_ref,
                  l4_ref, g4_ref, bm4_ref,
                  l5_ref, g5_ref, bm5_ref,
                  w1_ref, fb1_ref, w2_ref, fb2_ref, w3_ref, fb3_ref,
                  o_ref):
    f32 = jnp.float32

    def dot(a, b):
        return jnp.dot(a, b, preferred_element_type=f32)

    talls = []
    for i in range(_G):
        xi = x_ref[i]
        kv = dot(xi[0:252, :], bkv_ref[0])
        for di in range(1, 5):
            kv = kv + dot(xi[di:di + 252, :], bkv_ref[di])
        talls.append(dot(l1_ref[...], kv))

    acc_a = jnp.zeros((_G * 128, 1152), f32)
    acc_b = jnp.zeros((_G * 128, 1152), f32)
    for di in range(5):
        tcat = jnp.concatenate(
            [talls[i][di * 128:(di + 1) * 128, :] for i in range(_G)], axis=0)
        acc_a = acc_a + dot(tcat[:, 0:128], g1h0_ref[di])
        acc_b = acc_b + dot(tcat[:, 128:256], g1h1_ref[di])
    acc1 = jnp.concatenate(
        [acc_a[:, 0:896],
         acc_a[:, 896:1152] + acc_b[:, 0:256],
         acc_b[:, 256:1152]], axis=1) + jnp.concatenate([bm1_ref[...]] * _G,
                                                        axis=0)

    x2 = acc1
    t2s = [jnp.concatenate(
        [dot(l2_ref[di], x2[i * 128:(i + 1) * 128, :]) for i in range(_G)],
        axis=0) for di in range(3)]
    chunks = []
    for c, s in enumerate(_S2_STARTS):
        ch = bm2_ref[:, c * 256:(c + 1) * 256]
        for di in range(3):
            ch = ch + dot(t2s[di][:, s * 16:s * 16 + 1024], g2c_ref[di, c])
        chunks.append(ch)
    acc2 = jnp.concatenate(chunks, axis=1)

    x3 = acc2
    acc3 = bm3_ref[...]
    for di in range(3):
        t3 = dot(l3_ref[di], x3)
        acc3 = acc3 + dot(t3, g3_ref[di])

    x4 = acc3
    acc4 = bm4_ref[...]
    for di in range(3):
        t4 = dot(l4_ref[di], x4)
        acc4 = acc4 + dot(t4, g4_ref[di])

    x5 = acc4
    acc5 = bm5_ref[...]
    for di in range(5):
        t5 = dot(l5_ref[di], x5)
        acc5 = acc5 + dot(t5, g5_ref[di])

    h = fb1_ref[...]
    for r in range(4):
        h = h + dot(acc5[r * _G:(r + 1) * _G, :], w1_ref[r])
    h = jnp.maximum(h, 0.0)
    h = jnp.maximum(dot(h, w2_ref[...]) + fb2_ref[...], 0.0)
    logits = dot(h, w3_ref[...]) + fb3_ref[...]
    m = jnp.max(logits, axis=-1, keepdims=True)
    lse = m + jnp.log(jnp.sum(jnp.exp(logits - m), axis=-1, keepdims=True))
    o_ref[...] = logits - lse


def _blockdiag(lm, rp, cp):
    k, r, c = lm.shape
    lmp = jnp.pad(lm, ((0, 0), (0, rp - r), (0, cp - c)))
    eye = jnp.eye(_G, dtype=lm.dtype)
    return jnp.einsum('ij,krc->kirjc', eye, lmp).reshape(k, _G * rp, _G * cp)


def _pmajor_g(g, k, cin, s, sp, co_n, hp, hpp):
    g = g.reshape(k, cin, s, co_n * hp).transpose(0, 2, 1, 3)
    g = jnp.pad(g, ((0, 0), (0, sp - s), (0, 0), (0, 0)))
    g = g.reshape(k, sp * cin, co_n, hp).transpose(0, 1, 3, 2)
    g = jnp.pad(g, ((0, 0), (0, 0), (0, hpp - hp), (0, 0)))
    return g.reshape(k, sp * cin, hpp * co_n)


def _pmajor_bm(bm, co_n, hp, hpp, rp):
    b = bm.reshape(hp, co_n, hp).transpose(0, 2, 1)
    b = jnp.pad(b, ((0, rp - hp), (0, hpp - hp), (0, 0)))
    return b.reshape(rp, hpp * co_n)


def kernel(image, bkv, l1, g1, bm1, l2, g2, bm2, l3, g3, bm3,
           l4, g4, bm4, l5, g5, bm5, w1p, fb1, w2, fb2, w3, fb3):
    n = image.shape[0]
    assert n % _G == 0

    x = image.reshape(n, 256, 256)
    bkvp = jnp.pad(bkv, ((0, 0), (0, 0), (0, 4)))
    l1a = jnp.pad(l1, ((0, 0), (0, 4), (0, 0))).reshape(640, 252)

    g1r = jnp.pad(g1, ((0, 0), (0, 4), (0, 0)))
    g1r = g1r.reshape(5, 256, 16, 124).transpose(0, 1, 3, 2)
    g1r = jnp.pad(g1r, ((0, 0), (0, 0), (0, 4), (0, 0))).reshape(5, 256, 2048)
    g1h0 = g1r[:, 0:128, 0:1152]
    g1h1 = g1r[:, 128:256, 896:2048]
    bm1g = _pmajor_bm(bm1, 16, 124, 128, 128)

    l2p = jnp.pad(l2, ((0, 0), (0, 3), (0, 4)))
    g2r = _pmajor_g(g2, 3, 16, 124, 128, 16, 61, 64)
    g2c = jnp.stack(
        [g2r[:, s * 16:s * 16 + 1024, c * 256:(c + 1) * 256]
         for c, s in enumerate(_S2_STARTS)], axis=1)
    bm2g = _pmajor_bm(bm2, 16, 61, 64, 64)
    bm2g = jnp.tile(bm2g, (_G, 1))

    l3b = _blockdiag(l3, 32, 64)
    g3r = _pmajor_g(g3, 3, 16, 61, 64, 16, 29, 32)
    bm3g = jnp.tile(_pmajor_bm(bm3, 16, 29, 32, 32), (_G, 1))

    l4b = _blockdiag(l4, 16, 32)
    g4r = _pmajor_g(g4, 3, 16, 29, 32, 16, 13, 16)
    bm4g = jnp.tile(_pmajor_bm(bm4, 16, 13, 16, 16), (_G, 1))

    l5pad = jnp.pad(l5, ((0, 0), (0, 0), (0, 3)))
    eyeg = jnp.eye(_G, dtype=l5.dtype)
    l5b = jnp.einsum('ij,krc->krijc', eyeg, l5pad).reshape(
        5, 4 * _G, 16 * _G)
    g5r = _pmajor_g(g5, 5, 16, 13, 16, 16, 4, 4)
    bm5g = jnp.repeat(bm5.reshape(4, 16, 4).transpose(0, 2, 1).reshape(4, 64),
                      _G, axis=0)

    w1f = w1p.reshape(4, 16, 4, 128).transpose(0, 2, 1, 3).reshape(4, 64, 128)

    out = pl.pallas_call(
        _fused_kernel,
        out_shape=jax.ShapeDtypeStruct((n // _G, _G, 2), jnp.float32),
        grid=(n // _G,),
        in_specs=[pl.BlockSpec((_G, 256, 256), lambda i: (i, 0, 0))]
                 + [_VMEM] * 23,
        out_specs=pl.BlockSpec((None, _G, 2), lambda i: (i, 0, 0)),
        compiler_params=pltpu.CompilerParams(
            dimension_semantics=("parallel",),
            vmem_limit_bytes=62 * 1024 * 1024),
    )(x, bkvp, l1a, g1h0, g1h1, bm1g, l2p, g2c, bm2g, l3b, g3r, bm3g,
      l4b, g4r, bm4g, l5b, g5r, bm5g, w1f, fb1, w2, fb2, w3, fb3)
    return out.reshape(n, 2)


# speedup vs baseline: 1.0306x; 1.0306x over previous
import jax
import jax.numpy as jnp
from jax.experimental import pallas as pl
from jax.experimental.pallas import tpu as pltpu

_G = 8
_S2_STARTS = (0, 24, 48, 64)

_VMEM = pl.BlockSpec(memory_space=pltpu.MemorySpace.VMEM)


def _fused_kernel(x_ref, bkv_ref, l1_ref, g1_ref, bm1_ref,
                  l2_ref, g2c_ref, bm2_ref,
                  l3_ref, g3_ref, bm3_ref,
                  l4_ref, g4_ref, bm4_ref,
                  l5_ref, g5_ref, bm5_ref,
                  w1_ref, fb1_ref, w2_ref, fb2_ref, w3_ref, fb3_ref,
                  o_ref):
    f32 = jnp.float32

    def dot(a, b):
        return jnp.dot(a, b, preferred_element_type=f32)

    talls = []
    for i in range(_G):
        xi = x_ref[i]
        kv = dot(xi[0:252, :], bkv_ref[0])
        for di in range(1, 5):
            kv = kv + dot(xi[di:di + 252, :], bkv_ref[di])
        talls.append(dot(l1_ref[...], kv))

    tcat = jnp.concatenate(
        [jnp.concatenate([talls[i][di * 128:(di + 1) * 128, :]
                          for i in range(_G)], axis=0)
         for di in range(5)], axis=1)
    acc1 = dot(tcat, g1_ref[...]) \
        + jnp.concatenate([bm1_ref[...]] * _G, axis=0)

    x2 = acc1
    chunks = [bm2_ref[:, c * 256:(c + 1) * 256] for c in range(4)]
    for di in range(3):
        t2 = jnp.concatenate(
            [dot(l2_ref[di], x2[i * 128:(i + 1) * 128, :]) for i in range(_G)],
            axis=0)
        for c, s in enumerate(_S2_STARTS):
            chunks[c] = chunks[c] + dot(t2[:, s * 16:s * 16 + 1024],
                                        g2c_ref[di, c])
    acc2 = jnp.concatenate(chunks, axis=1)

    x3 = acc2
    acc3 = bm3_ref[...]
    for di in range(3):
        t3 = dot(l3_ref[di], x3)
        acc3 = acc3 + dot(t3, g3_ref[di])

    x4 = acc3
    acc4 = bm4_ref[...]
    for di in range(3):
        t4 = dot(l4_ref[di], x4)
        acc4 = acc4 + dot(t4, g4_ref[di])

    x5 = acc4
    acc5 = bm5_ref[...]
    for di in range(5):
        t5 = dot(l5_ref[di], x5)
        acc5 = acc5 + dot(t5, g5_ref[di])

    h = fb1_ref[...]
    for r in range(4):
        h = h + dot(acc5[r * _G:(r + 1) * _G, :], w1_ref[r])
    h = jnp.maximum(h, 0.0)
    h = jnp.maximum(dot(h, w2_ref[...]) + fb2_ref[...], 0.0)
    logits = dot(h, w3_ref[...]) + fb3_ref[...]
    m = jnp.max(logits, axis=-1, keepdims=True)
    lse = m + jnp.log(jnp.sum(jnp.exp(logits - m), axis=-1, keepdims=True))
    o_ref[...] = logits - lse


def _blockdiag(lm, rp, cp):
    k, r, c = lm.shape
    lmp = jnp.pad(lm, ((0, 0), (0, rp - r), (0, cp - c)))
    eye = jnp.eye(_G, dtype=lm.dtype)
    return jnp.einsum('ij,krc->kirjc', eye, lmp).reshape(k, _G * rp, _G * cp)


def _pmajor_g(g, k, cin, s, sp, co_n, hp, hpp):
    g = g.reshape(k, cin, s, co_n * hp).transpose(0, 2, 1, 3)
    g = jnp.pad(g, ((0, 0), (0, sp - s), (0, 0), (0, 0)))
    g = g.reshape(k, sp * cin, co_n, hp).transpose(0, 1, 3, 2)
    g = jnp.pad(g, ((0, 0), (0, 0), (0, hpp - hp), (0, 0)))
    return g.reshape(k, sp * cin, hpp * co_n)


def _pmajor_bm(bm, co_n, hp, hpp, rp):
    b = bm.reshape(hp, co_n, hp).transpose(0, 2, 1)
    b = jnp.pad(b, ((0, rp - hp), (0, hpp - hp), (0, 0)))
    return b.reshape(rp, hpp * co_n)


def kernel(image, bkv, l1, g1, bm1, l2, g2, bm2, l3, g3, bm3,
           l4, g4, bm4, l5, g5, bm5, w1p, fb1, w2, fb2, w3, fb3):
    n = image.shape[0]
    assert n % _G == 0
    one = fb3[0, 0] * 0.0 + 1.0

    x = image.reshape(n, 256, 256)
    bkvp = jnp.pad(bkv, ((0, 0), (0, 0), (0, 4)))
    l1a = jnp.pad(l1, ((0, 0), (0, 4), (0, 0))).reshape(640, 252)

    g1r = jnp.pad(g1, ((0, 0), (0, 4), (0, 0)))
    g1r = g1r.reshape(5, 256, 16, 124).transpose(0, 1, 3, 2)
    g1r = jnp.pad(g1r, ((0, 0), (0, 0), (0, 4), (0, 0)))
    g1s = (g1r * one).reshape(1280, 2048)
    bm1g = _pmajor_bm(bm1, 16, 124, 128, 128) * one

    l2p = jnp.pad(l2, ((0, 0), (0, 3), (0, 4)))
    g2r = _pmajor_g(g2, 3, 16, 124, 128, 16, 61, 64)
    g2c = jnp.stack(
        [g2r[:, s * 16:s * 16 + 1024, c * 256:(c + 1) * 256]
         for c, s in enumerate(_S2_STARTS)], axis=1) * one
    bm2g = jnp.tile(_pmajor_bm(bm2, 16, 61, 64, 64), (_G, 1)) * one

    l3b = _blockdiag(l3, 32, 64) * one
    g3r = _pmajor_g(g3, 3, 16, 61, 64, 16, 29, 32) * one
    bm3g = jnp.tile(_pmajor_bm(bm3, 16, 29, 32, 32), (_G, 1)) * one

    l4b = _blockdiag(l4, 16, 32) * one
    g4r = _pmajor_g(g4, 3, 16, 29, 32, 16, 13, 16) * one
    bm4g = jnp.tile(_pmajor_bm(bm4, 16, 13, 16, 16), (_G, 1)) * one

    l5pad = jnp.pad(l5, ((0, 0), (0, 0), (0, 3)))
    eyeg = jnp.eye(_G, dtype=l5.dtype)
    l5b = jnp.einsum('ij,krc->krijc', eyeg, l5pad).reshape(
        5, 4 * _G, 16 * _G) * one
    g5r = _pmajor_g(g5, 5, 16, 13, 16, 16, 4, 4) * one
    bm5g = jnp.repeat(bm5.reshape(4, 16, 4).transpose(0, 2, 1).reshape(4, 64),
                      _G, axis=0) * one

    w1f = w1p.reshape(4, 16, 4, 128).transpose(0, 2, 1, 3).reshape(
        4, 64, 128) * one

    out = pl.pallas_call(
        _fused_kernel,
        out_shape=jax.ShapeDtypeStruct((n // _G, _G, 2), jnp.float32),
        grid=(n // _G,),
        in_specs=[pl.BlockSpec((_G, 256, 256), lambda i: (i, 0, 0))]
                 + [_VMEM] * 22,
        out_specs=pl.BlockSpec((None, _G, 2), lambda i: (i, 0, 0)),
        compiler_params=pltpu.CompilerParams(
            dimension_semantics=("parallel",),
            vmem_limit_bytes=62 * 1024 * 1024),
    )(x, bkvp, l1a, g1s, bm1g, l2p, g2c, bm2g, l3b, g3r, bm3g,
      l4b, g4r, bm4g, l5b, g5r, bm5g, w1f, fb1, w2, fb2, w3, fb3)
    return out.reshape(n, 2)


# speedup vs baseline: 1.4454x; 1.4024x over previous
import jax
import jax.numpy as jnp
from jax.experimental import pallas as pl
from jax.experimental.pallas import tpu as pltpu

_G = 4
_CDT = jnp.float32

_VMEM = pl.BlockSpec(memory_space=pltpu.MemorySpace.VMEM)


def _fused_kernel(x_ref, bkv_ref, l1_ref, g1_ref, bm1_ref,
                  l2_ref, g2_ref, bm2_ref,
                  l3_ref, g3_ref, bm3_ref,
                  l4_ref, g4_ref, bm4_ref,
                  l5_ref, g5_ref, bm5_ref,
                  w1_ref, fb1_ref, w2_ref, fb2_ref, w3_ref, fb3_ref,
                  o_ref):
    f32 = jnp.float32

    def dot(a, b):
        return jnp.dot(a, b, preferred_element_type=f32)

    talls = []
    for i in range(_G):
        xi = x_ref[i]
        kv = dot(xi[0:252, :], bkv_ref[0])
        for di in range(1, 5):
            kv = kv + dot(xi[di:di + 252, :], bkv_ref[di])
        talls.append(dot(l1_ref[...], kv.astype(_CDT)).astype(_CDT))

    tcat = jnp.concatenate(
        [jnp.concatenate([talls[i][di * 128:(di + 1) * 128, :]
                          for i in range(_G)], axis=0)
         for di in range(5)], axis=1)
    acc1 = bm1_ref[...].astype(f32) + dot(tcat, g1_ref[...])

    x2 = acc1.astype(_CDT)
    acc2 = bm2_ref[...].astype(f32)
    for di in range(3):
        t2 = jnp.concatenate(
            [dot(l2_ref[di], x2[i * 128:(i + 1) * 128, :]).astype(_CDT)
             for i in range(_G)], axis=0)
        acc2 = acc2 + dot(t2, g2_ref[di])

    x3 = acc2.astype(_CDT)
    acc3 = bm3_ref[...].astype(f32)
    for di in range(3):
        t3 = dot(l3_ref[di], x3)
        acc3 = acc3 + dot(t3.astype(_CDT), g3_ref[di])

    x4 = acc3.astype(_CDT)
    acc4 = bm4_ref[...].astype(f32)
    for di in range(3):
        t4 = dot(l4_ref[di], x4)
        acc4 = acc4 + dot(t4.astype(_CDT), g4_ref[di])

    x5 = acc4.astype(_CDT)
    acc5 = bm5_ref[...].astype(f32)
    for di in range(5):
        t5 = dot(l5_ref[di], x5)
        acc5 = acc5 + dot(t5.astype(_CDT), g5_ref[di])

    a5 = acc5.astype(_CDT)
    h = fb1_ref[...]
    for r in range(4):
        h = h + dot(a5[r * _G:(r + 1) * _G, :], w1_ref[r])
    h = jnp.maximum(h, 0.0)
    h = jnp.maximum(dot(h.astype(_CDT), w2_ref[...]) + fb2_ref[...], 0.0)
    logits = dot(h.astype(_CDT), w3_ref[...]) + fb3_ref[...]
    m = jnp.max(logits, axis=-1, keepdims=True)
    lse = m + jnp.log(jnp.sum(jnp.exp(logits - m), axis=-1, keepdims=True))
    o_ref[...] = logits - lse


def _blockdiag(lm, rp, cp):
    k, r, c = lm.shape
    lmp = jnp.pad(lm, ((0, 0), (0, rp - r), (0, cp - c)))
    eye = jnp.eye(_G, dtype=lm.dtype)
    return jnp.einsum('ij,krc->kirjc', eye, lmp).reshape(k, _G * rp, _G * cp)


def _tile_rows(bm, rp):
    return jnp.tile(jnp.pad(bm, ((0, rp - bm.shape[0]), (0, 0))), (_G, 1))


def kernel(image, bkv, l1, g1, bm1, l2, g2, bm2, l3, g3, bm3,
           l4, g4, bm4, l5, g5, bm5, w1p, fb1, w2, fb2, w3, fb3):
    n = image.shape[0]
    assert n % _G == 0
    bf = _CDT

    x = image.reshape(n, 256, 256).astype(bf)
    bkvp = jnp.pad(bkv, ((0, 0), (0, 0), (0, 4))).astype(bf)
    l1a = jnp.pad(l1, ((0, 0), (0, 4), (0, 0))).reshape(640, 252).astype(bf)
    g1p = jnp.pad(g1, ((0, 0), (0, 4), (0, 0))).reshape(
        1280, 1984).astype(bf)
    bm1g = jnp.tile(jnp.pad(bm1, ((0, 4), (0, 0))), (_G, 1)).astype(bf)

    l2p = jnp.pad(l2, ((0, 0), (0, 3), (0, 4))).astype(bf)
    g2b = g2.astype(bf)
    bm2g = _tile_rows(bm2, 64).astype(bf)

    l3b = _blockdiag(l3, 32, 64).astype(bf)
    g3b = g3.astype(bf)
    bm3g = _tile_rows(bm3, 32).astype(bf)

    l4b = _blockdiag(l4, 16, 32).astype(bf)
    g4b = g4.astype(bf)
    bm4g = _tile_rows(bm4, 16).astype(bf)

    l5pad = jnp.pad(l5, ((0, 0), (0, 0), (0, 3)))
    eyeg = jnp.eye(_G, dtype=l5.dtype)
    l5b = jnp.einsum('ij,krc->krijc', eyeg, l5pad).reshape(
        5, 4 * _G, 16 * _G).astype(bf)
    g5b = g5.astype(bf)
    bm5g = jnp.repeat(bm5, _G, axis=0).astype(bf)

    w1f = w1p.astype(bf)
    w2b = w2.astype(bf)
    w3b = w3.astype(bf)

    out = pl.pallas_call(
        _fused_kernel,
        out_shape=jax.ShapeDtypeStruct((n // _G, _G, 2), jnp.float32),
        grid=(n // _G,),
        in_specs=[pl.BlockSpec((_G, 256, 256), lambda i: (i, 0, 0))]
                 + [_VMEM] * 22,
        out_specs=pl.BlockSpec((None, _G, 2), lambda i: (i, 0, 0)),
        compiler_params=pltpu.CompilerParams(
            dimension_semantics=("parallel",),
            vmem_limit_bytes=62 * 1024 * 1024),
    )(x, bkvp, l1a, g1p, bm1g, l2p, g2b, bm2g, l3b, g3b, bm3g,
      l4b, g4b, bm4g, l5b, g5b, bm5g, w1f, fb1, w2b, fb2, w3b, fb3)
    return out.reshape(n, 2)
